# baseline (speedup 1.0000x reference)
import functools

import jax
import jax.numpy as jnp
from jax import lax
from jax.experimental import pallas as pl
from jax.experimental.pallas import tpu as pltpu
from jax.experimental.pallas import tpu_sc as plsc

N = 10000
E = 320000
D = 128
NC = 2
NS = 16
NW = NC * NS
EPW = E // NW
CH = 125
NCHUNK = EPW // CH
NPAD = 10240
ROWS_PER_TILE = NPAD // NS
DW = 16

_mesh = plsc.VectorSubcoreMesh(
    core_axis_name="c", subcore_axis_name="s", num_cores=NC, num_subcores=NS)


@functools.partial(
    pl.kernel,
    out_type=jax.ShapeDtypeStruct((NC, 16, NPAD), jnp.float32),
    mesh=_mesh,
    scratch_types=[
        pltpu.VMEM((CH,), jnp.int32),
        pltpu.VMEM((CH,), jnp.int32),
        pltpu.VMEM((CH,), jnp.int32),
        pltpu.VMEM((CH,), jnp.int32),
        pltpu.VMEM((128,), jnp.float32),
        pltpu.VMEM((640,), jnp.float32),
        pltpu.VMEM_SHARED((NPAD,), jnp.float32),
        pltpu.VMEM_SHARED((NPAD,), jnp.float32),
        pltpu.SemaphoreType.DMA,
        pltpu.SemaphoreType.DMA,
        pltpu.SemaphoreType.DMA,
        pltpu.SemaphoreType.DMA,
    ])
def _deg_kernel(src_hbm, dst_hbm, out_hbm, sbA, sbB, dbA, dbB, ones_v, zbuf,
                dsrc_sh, ddst_sh, semA, semB, semSA, semSB):
    c = lax.axis_index("c")
    s = lax.axis_index("s")
    w = c * NS + s

    z16 = jnp.zeros((16,), jnp.float32)
    o16 = jnp.ones((16,), jnp.float32)

    def fill(i, _):
        zbuf[pl.ds(i * 16, 16)] = z16
        return 0
    lax.fori_loop(0, 40, fill, 0)

    def fill_ones(i, _):
        ones_v[pl.ds(i * 16, 16)] = o16
        return 0
    lax.fori_loop(0, 8, fill_ones, 0)
    ones = ones_v.at[pl.ds(0, CH)]

    pltpu.sync_copy(zbuf, dsrc_sh.at[pl.ds(s * 640, 640)])
    pltpu.sync_copy(zbuf, ddst_sh.at[pl.ds(s * 640, 640)])
    plsc.subcore_barrier()

    pltpu.sync_copy(src_hbm.at[w, 0, 0], sbA)
    pltpu.sync_copy(dst_hbm.at[w, 0, 0], dbA)
    pltpu.async_copy(src_hbm.at[w, 1, 0], sbB, semB)
    pltpu.async_copy(dst_hbm.at[w, 1, 0], dbB, semB)

    def body(k, _):
        j0 = k * 2
        pltpu.async_copy(ones, dsrc_sh.at[sbA], semSA, add=True)
        pltpu.async_copy(ones, ddst_sh.at[dbA], semSA, add=True)
        pltpu.make_async_copy(src_hbm.at[w, j0 + 1, 0], sbB, semB).wait()
        pltpu.make_async_copy(dst_hbm.at[w, j0 + 1, 0], dbB, semB).wait()
        pltpu.async_copy(ones, dsrc_sh.at[sbB], semSB, add=True)
        pltpu.async_copy(ones, ddst_sh.at[dbB], semSB, add=True)
        pltpu.make_async_copy(ones, dsrc_sh.at[sbA], semSA).wait()
        pltpu.make_async_copy(ones, ddst_sh.at[dbA], semSA).wait()

        @pl.when(k < NCHUNK // 2 - 1)
        def _():
            pltpu.async_copy(src_hbm.at[w, j0 + 2, 0], sbA, semA)
            pltpu.async_copy(dst_hbm.at[w, j0 + 2, 0], dbA, semA)
        pltpu.make_async_copy(ones, dsrc_sh.at[sbB], semSB).wait()
        pltpu.make_async_copy(ones, ddst_sh.at[dbB], semSB).wait()

        @pl.when(k < NCHUNK // 2 - 1)
        def _():
            pltpu.async_copy(src_hbm.at[w, j0 + 3, 0], sbB, semB)
            pltpu.async_copy(dst_hbm.at[w, j0 + 3, 0], dbB, semB)
            pltpu.make_async_copy(src_hbm.at[w, j0 + 2, 0], sbA, semA).wait()
            pltpu.make_async_copy(dst_hbm.at[w, j0 + 2, 0], dbA, semA).wait()
        return 0
    lax.fori_loop(0, NCHUNK // 2, body, 0)
    plsc.subcore_barrier()

    pltpu.sync_copy(dsrc_sh.at[pl.ds(s * 640, 640)],
                    out_hbm.at[c, 0, pl.ds(s * 640, 640)])
    pltpu.sync_copy(ddst_sh.at[pl.ds(s * 640, 640)],
                    out_hbm.at[c, 8, pl.ds(s * 640, 640)])


@functools.partial(
    pl.kernel,
    out_type=jax.ShapeDtypeStruct((NC, NPAD, D), jnp.float32),
    mesh=_mesh,
    scratch_types=[
        pltpu.VMEM((CH,), jnp.int32),
        pltpu.VMEM((CH,), jnp.int32),
        pltpu.VMEM((CH,), jnp.int32),
        pltpu.VMEM((CH,), jnp.int32),
        pltpu.VMEM((CH,), jnp.int32),
        pltpu.VMEM((CH,), jnp.int32),
        pltpu.VMEM((CH,), jnp.int32),
        pltpu.VMEM((CH,), jnp.int32),
        pltpu.VMEM((CH, D), jnp.float32),
        pltpu.VMEM((CH, D), jnp.float32),
        pltpu.VMEM((16, D), jnp.float32),
        pltpu.VMEM_SHARED((NPAD, D), jnp.float32),
        pltpu.SemaphoreType.DMA,
        pltpu.SemaphoreType.DMA,
        pltpu.SemaphoreType.DMA,
        pltpu.SemaphoreType.DMA,
        pltpu.SemaphoreType.DMA,
    ])
def _msg_kernel(h_hbm, src_hbm, dst_hbm, out_hbm, sA0, sA1, dA0, dA1,
                sB0, sB1, dB0, dB1, buf0, buf1, zbuf, acc_sh,
                semA, semB, semg0, semg1, semz):
    c = lax.axis_index("c")
    s = lax.axis_index("s")
    w = c * NS + s

    z16 = jnp.zeros((16,), jnp.float32)

    def fill(r, _):
        for k in range(D // 16):
            zbuf[r, pl.ds(k * 16, 16)] = z16
        return 0
    lax.fori_loop(0, 16, fill, 0)

    for i in range(ROWS_PER_TILE // 16):
        pltpu.async_copy(zbuf, acc_sh.at[pl.ds(s * ROWS_PER_TILE + i * 16, 16)],
                         semz)
    pltpu.sync_copy(src_hbm.at[w, 0, 0], sA0)
    pltpu.sync_copy(dst_hbm.at[w, 0, 0], dA0)
    pltpu.sync_copy(src_hbm.at[w, 1, 0], sA1)
    pltpu.sync_copy(dst_hbm.at[w, 1, 0], dA1)
    pltpu.async_copy(src_hbm.at[w, 2, 0], sB0, semB)
    pltpu.async_copy(dst_hbm.at[w, 2, 0], dB0, semB)
    pltpu.async_copy(src_hbm.at[w, 3, 0], sB1, semB)
    pltpu.async_copy(dst_hbm.at[w, 3, 0], dB1, semB)
    for i in range(ROWS_PER_TILE // 16):
        pltpu.make_async_copy(
            zbuf, acc_sh.at[pl.ds(s * ROWS_PER_TILE + i * 16, 16)], semz).wait()
    plsc.subcore_barrier()

    pltpu.async_copy(h_hbm.at[sA0], buf0, semg0)
    pltpu.async_copy(h_hbm.at[sA1], buf1, semg1)

    def wait_set(sbx, dbx, j0, sem):
        pltpu.make_async_copy(src_hbm.at[w, j0, 0], sbx, sem).wait()
        pltpu.make_async_copy(dst_hbm.at[w, j0, 0], dbx, sem).wait()

    def body(k, _):
        j0 = k * 4
        pltpu.make_async_copy(h_hbm.at[sA0], buf0, semg0).wait()
        pltpu.sync_copy(buf0, acc_sh.at[dA0], add=True)
        wait_set(sB0, dB0, j0 + 2, semB)
        wait_set(sB1, dB1, j0 + 3, semB)
        pltpu.async_copy(h_hbm.at[sB0], buf0, semg0)
        pltpu.make_async_copy(h_hbm.at[sA1], buf1, semg1).wait()
        pltpu.sync_copy(buf1, acc_sh.at[dA1], add=True)
        pltpu.async_copy(h_hbm.at[sB1], buf1, semg1)
        pltpu.async_copy(src_hbm.at[w, j0 + 4, 0], sA0, semA)
        pltpu.async_copy(dst_hbm.at[w, j0 + 4, 0], dA0, semA)
        pltpu.async_copy(src_hbm.at[w, j0 + 5, 0], sA1, semA)
        pltpu.async_copy(dst_hbm.at[w, j0 + 5, 0], dA1, semA)
        pltpu.make_async_copy(h_hbm.at[sB0], buf0, semg0).wait()
        pltpu.sync_copy(buf0, acc_sh.at[dB0], add=True)
        wait_set(sA0, dA0, j0 + 4, semA)
        wait_set(sA1, dA1, j0 + 5, semA)
        pltpu.async_copy(h_hbm.at[sA0], buf0, semg0)
        pltpu.make_async_copy(h_hbm.at[sB1], buf1, semg1).wait()
        pltpu.sync_copy(buf1, acc_sh.at[dB1], add=True)
        pltpu.async_copy(h_hbm.at[sA1], buf1, semg1)

        pltpu.async_copy(src_hbm.at[w, j0 + 6, 0], sB0, semB)
        pltpu.async_copy(dst_hbm.at[w, j0 + 6, 0], dB0, semB)
        pltpu.async_copy(src_hbm.at[w, j0 + 7, 0], sB1, semB)
        pltpu.async_copy(dst_hbm.at[w, j0 + 7, 0], dB1, semB)
        return 0
    lax.fori_loop(0, NCHUNK // 4 - 1, body, 0)

    j0 = NCHUNK - 4
    pltpu.make_async_copy(h_hbm.at[sA0], buf0, semg0).wait()
    pltpu.sync_copy(buf0, acc_sh.at[dA0], add=True)
    wait_set(sB0, dB0, j0 + 2, semB)
    wait_set(sB1, dB1, j0 + 3, semB)
    pltpu.async_copy(h_hbm.at[sB0], buf0, semg0)
    pltpu.make_async_copy(h_hbm.at[sA1], buf1, semg1).wait()
    pltpu.sync_copy(buf1, acc_sh.at[dA1], add=True)
    pltpu.async_copy(h_hbm.at[sB1], buf1, semg1)
    pltpu.make_async_copy(h_hbm.at[sB0], buf0, semg0).wait()
    pltpu.sync_copy(buf0, acc_sh.at[dB0], add=True)
    pltpu.make_async_copy(h_hbm.at[sB1], buf1, semg1).wait()
    pltpu.sync_copy(buf1, acc_sh.at[dB1], add=True)
    plsc.subcore_barrier()

    pltpu.sync_copy(acc_sh.at[pl.ds(s * ROWS_PER_TILE, ROWS_PER_TILE)],
                    out_hbm.at[c, pl.ds(s * ROWS_PER_TILE, ROWS_PER_TILE)])


_R = 1000


def _tc0_body(hraw_ref, deg_ref, h_ref, ns_ref, nd_ref):
    degs = deg_ref[...]
    d_out = degs[0, :, 0:1] + degs[1, :, 0:1]
    d_in = degs[0, :, 8:9] + degs[1, :, 8:9]
    ns = jnp.where(d_out > 0, lax.rsqrt(jnp.maximum(d_out, 1.0)), 0.0)
    nd = jnp.where(d_in > 0, lax.rsqrt(jnp.maximum(d_in, 1.0)), 0.0)
    h_ref[...] = hraw_ref[...] * ns
    ns_ref[...] = ns
    nd_ref[...] = nd


def _tc_mm_body(x_ref, w_ref, o_ref):
    o_ref[...] = jnp.dot(x_ref[...], w_ref[...],
                        preferred_element_type=jnp.float32)


def _tc_mid_body(p_ref, nd_ref, b_ref, w_ref, ns_ref, o_ref):
    z = (p_ref[0] + p_ref[1]) * nd_ref[...] + b_ref[...]
    h = jnp.maximum(z, 0.0)
    o_ref[...] = jnp.dot(h, w_ref[...],
                         preferred_element_type=jnp.float32) * ns_ref[...]


def _tc_fin_body(p_ref, nd_ref, b_ref, o_ref):
    o_ref[...] = (p_ref[0] + p_ref[1]) * nd_ref[...] + b_ref[...]


_tc_mm = pl.pallas_call(
    _tc_mm_body,
    grid=(N // _R,),
    in_specs=[
        pl.BlockSpec((_R, D), lambda j: (j, 0)),
        pl.BlockSpec((D, D), lambda j: (0, 0)),
    ],
    out_specs=pl.BlockSpec((_R, D), lambda j: (j, 0)),
    out_shape=jax.ShapeDtypeStruct((N, D), jnp.float32),
)

_tc0 = pl.pallas_call(
    _tc0_body,
    grid=(N // _R,),
    in_specs=[
        pl.BlockSpec((_R, D), lambda j: (j, 0)),
        pl.BlockSpec((NC, _R, 16), lambda j: (0, j, 0)),
    ],
    out_specs=[
        pl.BlockSpec((_R, D), lambda j: (j, 0)),
        pl.BlockSpec((_R, 1), lambda j: (j, 0)),
        pl.BlockSpec((_R, 1), lambda j: (j, 0)),
    ],
    out_shape=[
        jax.ShapeDtypeStruct((N, D), jnp.float32),
        jax.ShapeDtypeStruct((N, 1), jnp.float32),
        jax.ShapeDtypeStruct((N, 1), jnp.float32),
    ],
)

_tc_mid = pl.pallas_call(
    _tc_mid_body,
    grid=(N // _R,),
    in_specs=[
        pl.BlockSpec((NC, _R, D), lambda j: (0, j, 0)),
        pl.BlockSpec((_R, 1), lambda j: (j, 0)),
        pl.BlockSpec((1, D), lambda j: (0, 0)),
        pl.BlockSpec((D, D), lambda j: (0, 0)),
        pl.BlockSpec((_R, 1), lambda j: (j, 0)),
    ],
    out_specs=pl.BlockSpec((_R, D), lambda j: (j, 0)),
    out_shape=jax.ShapeDtypeStruct((N, D), jnp.float32),
)

_tc_fin = pl.pallas_call(
    _tc_fin_body,
    grid=(N // _R,),
    in_specs=[
        pl.BlockSpec((NC, _R, D), lambda j: (0, j, 0)),
        pl.BlockSpec((_R, 1), lambda j: (j, 0)),
        pl.BlockSpec((1, D), lambda j: (0, 0)),
    ],
    out_specs=pl.BlockSpec((_R, D), lambda j: (j, 0)),
    out_shape=jax.ShapeDtypeStruct((N, D), jnp.float32),
)


def kernel(x, edge_index, W_in, b_in, W_hid, b_hid, W_out, b_out):
    src3 = edge_index[0].astype(jnp.int32).reshape(NW, NCHUNK, 1, CH)
    dst3 = edge_index[1].astype(jnp.int32).reshape(NW, NCHUNK, 1, CH)
    deg = _deg_kernel(src3, dst3).transpose(0, 2, 1)
    hraw = _tc_mm(x, W_in)
    h1, ns, nd = _tc0(hraw, deg)
    p1 = _msg_kernel(h1, src3, dst3)
    h2 = _tc_mid(p1, nd, b_in.reshape(1, D), W_hid, ns)
    p2 = _msg_kernel(h2, src3, dst3)
    h3 = _tc_mid(p2, nd, b_hid.reshape(1, D), W_out, ns)
    p3 = _msg_kernel(h3, src3, dst3)
    return _tc_fin(p3, nd, b_out.reshape(1, D))


# speedup vs baseline: 1.0126x; 1.0126x over previous
import functools

import jax
import jax.numpy as jnp
from jax import lax
from jax.experimental import pallas as pl
from jax.experimental.pallas import tpu as pltpu
from jax.experimental.pallas import tpu_sc as plsc

N = 10000
E = 320000
D = 128
NC = 2
NS = 16
NW = NC * NS
EPW = E // NW
CH = 80
NCHUNK = EPW // CH
NPAD = 10240
ROWS_PER_TILE = NPAD // NS
DW = 16

_mesh = plsc.VectorSubcoreMesh(
    core_axis_name="c", subcore_axis_name="s", num_cores=NC, num_subcores=NS)


@functools.partial(
    pl.kernel,
    out_type=jax.ShapeDtypeStruct((NC, 16, NPAD), jnp.float32),
    mesh=_mesh,
    scratch_types=[
        pltpu.VMEM((CH,), jnp.int32),
        pltpu.VMEM((CH,), jnp.int32),
        pltpu.VMEM((CH,), jnp.int32),
        pltpu.VMEM((CH,), jnp.int32),
        pltpu.VMEM((128,), jnp.float32),
        pltpu.VMEM((640,), jnp.float32),
        pltpu.VMEM_SHARED((NPAD,), jnp.float32),
        pltpu.VMEM_SHARED((NPAD,), jnp.float32),
        pltpu.SemaphoreType.DMA,
        pltpu.SemaphoreType.DMA,
        pltpu.SemaphoreType.DMA,
        pltpu.SemaphoreType.DMA,
    ])
def _deg_kernel(src_hbm, dst_hbm, out_hbm, sbA, sbB, dbA, dbB, ones_v, zbuf,
                dsrc_sh, ddst_sh, semA, semB, semSA, semSB):
    c = lax.axis_index("c")
    s = lax.axis_index("s")
    w = c * NS + s

    z16 = jnp.zeros((16,), jnp.float32)
    o16 = jnp.ones((16,), jnp.float32)

    def fill(i, _):
        zbuf[pl.ds(i * 16, 16)] = z16
        return 0
    lax.fori_loop(0, 40, fill, 0)

    def fill_ones(i, _):
        ones_v[pl.ds(i * 16, 16)] = o16
        return 0
    lax.fori_loop(0, 8, fill_ones, 0)
    ones = ones_v.at[pl.ds(0, CH)]

    pltpu.sync_copy(zbuf, dsrc_sh.at[pl.ds(s * 640, 640)])
    pltpu.sync_copy(zbuf, ddst_sh.at[pl.ds(s * 640, 640)])
    plsc.subcore_barrier()

    pltpu.sync_copy(src_hbm.at[w, 0, 0], sbA)
    pltpu.sync_copy(dst_hbm.at[w, 0, 0], dbA)
    pltpu.async_copy(src_hbm.at[w, 1, 0], sbB, semB)
    pltpu.async_copy(dst_hbm.at[w, 1, 0], dbB, semB)

    def body(k, _):
        j0 = k * 2
        pltpu.async_copy(ones, dsrc_sh.at[sbA], semSA, add=True)
        pltpu.async_copy(ones, ddst_sh.at[dbA], semSA, add=True)
        pltpu.make_async_copy(src_hbm.at[w, j0 + 1, 0], sbB, semB).wait()
        pltpu.make_async_copy(dst_hbm.at[w, j0 + 1, 0], dbB, semB).wait()
        pltpu.async_copy(ones, dsrc_sh.at[sbB], semSB, add=True)
        pltpu.async_copy(ones, ddst_sh.at[dbB], semSB, add=True)
        pltpu.make_async_copy(ones, dsrc_sh.at[sbA], semSA).wait()
        pltpu.make_async_copy(ones, ddst_sh.at[dbA], semSA).wait()
        pltpu.async_copy(src_hbm.at[w, j0 + 2, 0], sbA, semA)
        pltpu.async_copy(dst_hbm.at[w, j0 + 2, 0], dbA, semA)
        pltpu.make_async_copy(ones, dsrc_sh.at[sbB], semSB).wait()
        pltpu.make_async_copy(ones, ddst_sh.at[dbB], semSB).wait()

        @pl.when(k < NCHUNK // 2 - 1)
        def _():
            pltpu.async_copy(src_hbm.at[w, j0 + 3, 0], sbB, semB)
            pltpu.async_copy(dst_hbm.at[w, j0 + 3, 0], dbB, semB)
        pltpu.make_async_copy(src_hbm.at[w, j0 + 2, 0], sbA, semA).wait()
        pltpu.make_async_copy(dst_hbm.at[w, j0 + 2, 0], dbA, semA).wait()
        return 0
    lax.fori_loop(0, NCHUNK // 2, body, 0)
    pltpu.sync_copy(ones, dsrc_sh.at[sbA], add=True)
    pltpu.sync_copy(ones, ddst_sh.at[dbA], add=True)
    plsc.subcore_barrier()

    pltpu.sync_copy(dsrc_sh.at[pl.ds(s * 640, 640)],
                    out_hbm.at[c, 0, pl.ds(s * 640, 640)])
    pltpu.sync_copy(ddst_sh.at[pl.ds(s * 640, 640)],
                    out_hbm.at[c, 8, pl.ds(s * 640, 640)])


@functools.partial(
    pl.kernel,
    out_type=jax.ShapeDtypeStruct((NC, NPAD, D), jnp.float32),
    mesh=_mesh,
    scratch_types=[
        pltpu.VMEM((CH,), jnp.int32),
        pltpu.VMEM((CH,), jnp.int32),
        pltpu.VMEM((CH,), jnp.int32),
        pltpu.VMEM((CH,), jnp.int32),
        pltpu.VMEM((CH,), jnp.int32),
        pltpu.VMEM((CH,), jnp.int32),
        pltpu.VMEM((CH, D), jnp.float32),
        pltpu.VMEM((CH, D), jnp.float32),
        pltpu.VMEM((CH, D), jnp.float32),
        pltpu.VMEM((16, D), jnp.float32),
        pltpu.VMEM_SHARED((NPAD, D), jnp.float32),
        pltpu.SemaphoreType.DMA,
        pltpu.SemaphoreType.DMA,
        pltpu.SemaphoreType.DMA,
        pltpu.SemaphoreType.DMA,
        pltpu.SemaphoreType.DMA,
        pltpu.SemaphoreType.DMA,
        pltpu.SemaphoreType.DMA,
        pltpu.SemaphoreType.DMA,
        pltpu.SemaphoreType.DMA,
        pltpu.SemaphoreType.DMA,
    ])
def _msg_kernel(h_hbm, src_hbm, dst_hbm, out_hbm,
                sb0, sb1, sb2, db0, db1, db2, buf0, buf1, buf2, zbuf, acc_sh,
                semg0, semg1, semg2, semIs0, semIs1, semIs2,
                semId0, semId1, semId2, semz):
    c = lax.axis_index("c")
    s = lax.axis_index("s")
    w = c * NS + s
    sb = [sb0, sb1, sb2]
    db = [db0, db1, db2]
    buf = [buf0, buf1, buf2]
    semg = [semg0, semg1, semg2]
    semIs = [semIs0, semIs1, semIs2]
    semId = [semId0, semId1, semId2]

    z16 = jnp.zeros((16,), jnp.float32)

    def fill(r, _):
        for k in range(D // 16):
            zbuf[r, pl.ds(k * 16, 16)] = z16
        return 0
    lax.fori_loop(0, 16, fill, 0)

    for i in range(ROWS_PER_TILE // 16):
        pltpu.async_copy(zbuf, acc_sh.at[pl.ds(s * ROWS_PER_TILE + i * 16, 16)],
                         semz)
    for i in range(3):
        pltpu.sync_copy(src_hbm.at[w, i, 0], sb[i])
        pltpu.sync_copy(dst_hbm.at[w, i, 0], db[i])
    pltpu.async_copy(h_hbm.at[sb0], buf0, semg0)
    pltpu.async_copy(h_hbm.at[sb1], buf1, semg1)
    for i in range(ROWS_PER_TILE // 16):
        pltpu.make_async_copy(
            zbuf, acc_sh.at[pl.ds(s * ROWS_PER_TILE + i * 16, 16)], semz).wait()
    plsc.subcore_barrier()

    NITER = (NCHUNK - 2) // 3

    def body(k, _):
        for i in range(3):
            j = 3 * k + i
            i2 = (i + 2) % 3
            pltpu.make_async_copy(h_hbm.at[sb[i]], buf[i], semg[i]).wait()
            if i < 2:
                pltpu.async_copy(src_hbm.at[w, j + 3, 0], sb[i], semIs[i])
            else:
                @pl.when(k < NITER - 1)
                def _():
                    pltpu.async_copy(src_hbm.at[w, j + 3, 0], sb[i], semIs[i])
            if i2 == 2:
                @pl.when(k > 0)
                def _():
                    pltpu.make_async_copy(
                        src_hbm.at[w, j + 2, 0], sb[i2], semIs[i2]).wait()
            else:
                pltpu.make_async_copy(
                    src_hbm.at[w, j + 2, 0], sb[i2], semIs[i2]).wait()
            pltpu.async_copy(h_hbm.at[sb[i2]], buf[i2], semg[i2])
            @pl.when(k > 0)
            def _():
                pltpu.make_async_copy(
                    dst_hbm.at[w, j, 0], db[i], semId[i]).wait()
            pltpu.sync_copy(buf[i], acc_sh.at[db[i]], add=True)
            if i < 2:
                pltpu.async_copy(dst_hbm.at[w, j + 3, 0], db[i], semId[i])
            else:
                @pl.when(k < NITER - 1)
                def _():
                    pltpu.async_copy(dst_hbm.at[w, j + 3, 0], db[i], semId[i])
        return 0
    lax.fori_loop(0, NITER, body, 0)

    jlast = NCHUNK - 2
    pltpu.make_async_copy(h_hbm.at[sb0], buf0, semg0).wait()
    pltpu.make_async_copy(dst_hbm.at[w, jlast, 0], db0, semId0).wait()
    pltpu.sync_copy(buf0, acc_sh.at[db0], add=True)
    pltpu.make_async_copy(h_hbm.at[sb1], buf1, semg1).wait()
    pltpu.make_async_copy(dst_hbm.at[w, jlast + 1, 0], db1, semId1).wait()
    pltpu.sync_copy(buf1, acc_sh.at[db1], add=True)
    plsc.subcore_barrier()

    pltpu.sync_copy(acc_sh.at[pl.ds(s * ROWS_PER_TILE, ROWS_PER_TILE)],
                    out_hbm.at[c, pl.ds(s * ROWS_PER_TILE, ROWS_PER_TILE)])


_R = 1000


def _tc0_body(hraw_ref, deg_ref, h_ref, ns_ref, nd_ref):
    degs = deg_ref[...]
    d_out = degs[0, :, 0:1] + degs[1, :, 0:1]
    d_in = degs[0, :, 8:9] + degs[1, :, 8:9]
    ns = jnp.where(d_out > 0, lax.rsqrt(jnp.maximum(d_out, 1.0)), 0.0)
    nd = jnp.where(d_in > 0, lax.rsqrt(jnp.maximum(d_in, 1.0)), 0.0)
    h_ref[...] = hraw_ref[...] * ns
    ns_ref[...] = ns
    nd_ref[...] = nd


def _tc_mm_body(x_ref, w_ref, o_ref):
    o_ref[...] = jnp.dot(x_ref[...], w_ref[...],
                        preferred_element_type=jnp.float32)


def _tc_mid_body(p_ref, nd_ref, b_ref, w_ref, ns_ref, o_ref):
    z = (p_ref[0] + p_ref[1]) * nd_ref[...] + b_ref[...]
    h = jnp.maximum(z, 0.0)
    o_ref[...] = jnp.dot(h, w_ref[...],
                         preferred_element_type=jnp.float32) * ns_ref[...]


def _tc_fin_body(p_ref, nd_ref, b_ref, o_ref):
    o_ref[...] = (p_ref[0] + p_ref[1]) * nd_ref[...] + b_ref[...]


_tc_mm = pl.pallas_call(
    _tc_mm_body,
    grid=(N // _R,),
    in_specs=[
        pl.BlockSpec((_R, D), lambda j: (j, 0)),
        pl.BlockSpec((D, D), lambda j: (0, 0)),
    ],
    out_specs=pl.BlockSpec((_R, D), lambda j: (j, 0)),
    out_shape=jax.ShapeDtypeStruct((N, D), jnp.float32),
)

_tc0 = pl.pallas_call(
    _tc0_body,
    grid=(N // _R,),
    in_specs=[
        pl.BlockSpec((_R, D), lambda j: (j, 0)),
        pl.BlockSpec((NC, _R, 16), lambda j: (0, j, 0)),
    ],
    out_specs=[
        pl.BlockSpec((_R, D), lambda j: (j, 0)),
        pl.BlockSpec((_R, 1), lambda j: (j, 0)),
        pl.BlockSpec((_R, 1), lambda j: (j, 0)),
    ],
    out_shape=[
        jax.ShapeDtypeStruct((N, D), jnp.float32),
        jax.ShapeDtypeStruct((N, 1), jnp.float32),
        jax.ShapeDtypeStruct((N, 1), jnp.float32),
    ],
)

_tc_mid = pl.pallas_call(
    _tc_mid_body,
    grid=(N // _R,),
    in_specs=[
        pl.BlockSpec((NC, _R, D), lambda j: (0, j, 0)),
        pl.BlockSpec((_R, 1), lambda j: (j, 0)),
        pl.BlockSpec((1, D), lambda j: (0, 0)),
        pl.BlockSpec((D, D), lambda j: (0, 0)),
        pl.BlockSpec((_R, 1), lambda j: (j, 0)),
    ],
    out_specs=pl.BlockSpec((_R, D), lambda j: (j, 0)),
    out_shape=jax.ShapeDtypeStruct((N, D), jnp.float32),
)

_tc_fin = pl.pallas_call(
    _tc_fin_body,
    grid=(N // _R,),
    in_specs=[
        pl.BlockSpec((NC, _R, D), lambda j: (0, j, 0)),
        pl.BlockSpec((_R, 1), lambda j: (j, 0)),
        pl.BlockSpec((1, D), lambda j: (0, 0)),
    ],
    out_specs=pl.BlockSpec((_R, D), lambda j: (j, 0)),
    out_shape=jax.ShapeDtypeStruct((N, D), jnp.float32),
)


def kernel(x, edge_index, W_in, b_in, W_hid, b_hid, W_out, b_out):
    src3 = edge_index[0].astype(jnp.int32).reshape(NW, NCHUNK, 1, CH)
    dst3 = edge_index[1].astype(jnp.int32).reshape(NW, NCHUNK, 1, CH)
    deg = _deg_kernel(src3, dst3).transpose(0, 2, 1)
    h1, ns, nd = _tc0(_tc_mm(x, W_in), deg)
    p1 = _msg_kernel(h1, src3, dst3)
    h2 = _tc_mid(p1, nd, b_in.reshape(1, D), W_hid, ns)
    p2 = _msg_kernel(h2, src3, dst3)
    h3 = _tc_mid(p2, nd, b_hid.reshape(1, D), W_out, ns)
    p3 = _msg_kernel(h3, src3, dst3)
    return _tc_fin(p3, nd, b_out.reshape(1, D))


# speedup vs baseline: 1.0515x; 1.0384x over previous
import functools

import jax
import jax.numpy as jnp
from jax import lax
from jax.experimental import pallas as pl
from jax.experimental.pallas import tpu as pltpu
from jax.experimental.pallas import tpu_sc as plsc

N = 10000
E = 320000
D = 128
NC = 2
NS = 16
NW = NC * NS
EPW = E // NW
CH = 80
NCHUNK = EPW // CH
NPAD = 10240
ROWS_PER_TILE = NPAD // NS
DW = 16

_mesh = plsc.VectorSubcoreMesh(
    core_axis_name="c", subcore_axis_name="s", num_cores=NC, num_subcores=NS)


CHD = 125
NCHD = EPW // CHD


@functools.partial(
    pl.kernel,
    out_type=jax.ShapeDtypeStruct((NC, 16, NPAD), jnp.float32),
    mesh=_mesh,
    scratch_types=[
        pltpu.VMEM((CHD,), jnp.int32),
        pltpu.VMEM((CHD,), jnp.int32),
        pltpu.VMEM((CHD,), jnp.int32),
        pltpu.VMEM((CHD,), jnp.int32),
        pltpu.VMEM((128,), jnp.float32),
        pltpu.VMEM((640,), jnp.float32),
        pltpu.VMEM_SHARED((NPAD,), jnp.float32),
        pltpu.VMEM_SHARED((NPAD,), jnp.float32),
        pltpu.SemaphoreType.DMA,
        pltpu.SemaphoreType.DMA,
        pltpu.SemaphoreType.DMA,
        pltpu.SemaphoreType.DMA,
    ])
def _deg_kernel(src_hbm, dst_hbm, out_hbm, sbA, sbB, dbA, dbB, ones_v, zbuf,
                dsrc_sh, ddst_sh, semA, semB, semSA, semSB):
    c = lax.axis_index("c")
    s = lax.axis_index("s")
    w = c * NS + s

    z16 = jnp.zeros((16,), jnp.float32)
    o16 = jnp.ones((16,), jnp.float32)

    def fill(i, _):
        zbuf[pl.ds(i * 16, 16)] = z16
        return 0
    lax.fori_loop(0, 40, fill, 0)

    def fill_ones(i, _):
        ones_v[pl.ds(i * 16, 16)] = o16
        return 0
    lax.fori_loop(0, 8, fill_ones, 0)
    ones = ones_v.at[pl.ds(0, CHD)]

    pltpu.sync_copy(zbuf, dsrc_sh.at[pl.ds(s * 640, 640)])
    pltpu.sync_copy(zbuf, ddst_sh.at[pl.ds(s * 640, 640)])
    plsc.subcore_barrier()

    pltpu.sync_copy(src_hbm.at[w, 0, 0], sbA)
    pltpu.sync_copy(dst_hbm.at[w, 0, 0], dbA)
    pltpu.async_copy(src_hbm.at[w, 1, 0], sbB, semB)
    pltpu.async_copy(dst_hbm.at[w, 1, 0], dbB, semB)

    def body(k, _):
        j0 = k * 2
        pltpu.async_copy(ones, dsrc_sh.at[sbA], semSA, add=True)
        pltpu.async_copy(ones, ddst_sh.at[dbA], semSA, add=True)
        pltpu.make_async_copy(src_hbm.at[w, j0 + 1, 0], sbB, semB).wait()
        pltpu.make_async_copy(dst_hbm.at[w, j0 + 1, 0], dbB, semB).wait()
        pltpu.async_copy(ones, dsrc_sh.at[sbB], semSB, add=True)
        pltpu.async_copy(ones, ddst_sh.at[dbB], semSB, add=True)
        pltpu.make_async_copy(ones, dsrc_sh.at[sbA], semSA).wait()
        pltpu.make_async_copy(ones, ddst_sh.at[dbA], semSA).wait()

        @pl.when(k < NCHD // 2 - 1)
        def _():
            pltpu.async_copy(src_hbm.at[w, j0 + 2, 0], sbA, semA)
            pltpu.async_copy(dst_hbm.at[w, j0 + 2, 0], dbA, semA)
        pltpu.make_async_copy(ones, dsrc_sh.at[sbB], semSB).wait()
        pltpu.make_async_copy(ones, ddst_sh.at[dbB], semSB).wait()

        @pl.when(k < NCHD // 2 - 1)
        def _():
            pltpu.async_copy(src_hbm.at[w, j0 + 3, 0], sbB, semB)
            pltpu.async_copy(dst_hbm.at[w, j0 + 3, 0], dbB, semB)
            pltpu.make_async_copy(src_hbm.at[w, j0 + 2, 0], sbA, semA).wait()
            pltpu.make_async_copy(dst_hbm.at[w, j0 + 2, 0], dbA, semA).wait()
        return 0
    lax.fori_loop(0, NCHD // 2, body, 0)
    plsc.subcore_barrier()

    pltpu.sync_copy(dsrc_sh.at[pl.ds(s * 640, 640)],
                    out_hbm.at[c, 0, pl.ds(s * 640, 640)])
    pltpu.sync_copy(ddst_sh.at[pl.ds(s * 640, 640)],
                    out_hbm.at[c, 8, pl.ds(s * 640, 640)])


@functools.partial(
    pl.kernel,
    out_type=jax.ShapeDtypeStruct((NC, NPAD, D), jnp.float32),
    mesh=_mesh,
    scratch_types=[
        pltpu.VMEM((CH,), jnp.int32),
        pltpu.VMEM((CH,), jnp.int32),
        pltpu.VMEM((CH,), jnp.int32),
        pltpu.VMEM((CH,), jnp.int32),
        pltpu.VMEM((CH,), jnp.int32),
        pltpu.VMEM((CH,), jnp.int32),
        pltpu.VMEM((CH, D), jnp.float32),
        pltpu.VMEM((CH, D), jnp.float32),
        pltpu.VMEM((CH, D), jnp.float32),
        pltpu.VMEM((16, D), jnp.float32),
        pltpu.VMEM_SHARED((NPAD, D), jnp.float32),
        pltpu.SemaphoreType.DMA,
        pltpu.SemaphoreType.DMA,
        pltpu.SemaphoreType.DMA,
        pltpu.SemaphoreType.DMA,
        pltpu.SemaphoreType.DMA,
        pltpu.SemaphoreType.DMA,
        pltpu.SemaphoreType.DMA,
        pltpu.SemaphoreType.DMA,
        pltpu.SemaphoreType.DMA,
        pltpu.SemaphoreType.DMA,
    ])
def _msg_kernel(h_hbm, src_hbm, dst_hbm, out_hbm,
                sb0, sb1, sb2, db0, db1, db2, buf0, buf1, buf2, zbuf, acc_sh,
                semg0, semg1, semg2, semIs0, semIs1, semIs2,
                semId0, semId1, semId2, semz):
    c = lax.axis_index("c")
    s = lax.axis_index("s")
    w = c * NS + s
    sb = [sb0, sb1, sb2]
    db = [db0, db1, db2]
    buf = [buf0, buf1, buf2]
    semg = [semg0, semg1, semg2]
    semIs = [semIs0, semIs1, semIs2]
    semId = [semId0, semId1, semId2]

    z16 = jnp.zeros((16,), jnp.float32)

    def fill(r, _):
        for k in range(D // 16):
            zbuf[r, pl.ds(k * 16, 16)] = z16
        return 0
    lax.fori_loop(0, 16, fill, 0)

    for i in range(ROWS_PER_TILE // 16):
        pltpu.async_copy(zbuf, acc_sh.at[pl.ds(s * ROWS_PER_TILE + i * 16, 16)],
                         semz)
    for i in range(3):
        pltpu.sync_copy(src_hbm.at[w, i, 0], sb[i])
        pltpu.sync_copy(dst_hbm.at[w, i, 0], db[i])
    pltpu.async_copy(h_hbm.at[sb0], buf0, semg0)
    pltpu.async_copy(h_hbm.at[sb1], buf1, semg1)
    for i in range(ROWS_PER_TILE // 16):
        pltpu.make_async_copy(
            zbuf, acc_sh.at[pl.ds(s * ROWS_PER_TILE + i * 16, 16)], semz).wait()
    plsc.subcore_barrier()

    NITER = (NCHUNK - 2) // 3

    def body(k, _):
        for i in range(3):
            j = 3 * k + i
            i2 = (i + 2) % 3
            pltpu.make_async_copy(h_hbm.at[sb[i]], buf[i], semg[i]).wait()
            if i < 2:
                pltpu.async_copy(src_hbm.at[w, j + 3, 0], sb[i], semIs[i])
            else:
                @pl.when(k < NITER - 1)
                def _():
                    pltpu.async_copy(src_hbm.at[w, j + 3, 0], sb[i], semIs[i])
            if i2 == 2:
                @pl.when(k > 0)
                def _():
                    pltpu.make_async_copy(
                        src_hbm.at[w, j + 2, 0], sb[i2], semIs[i2]).wait()
            else:
                pltpu.make_async_copy(
                    src_hbm.at[w, j + 2, 0], sb[i2], semIs[i2]).wait()
            pltpu.async_copy(h_hbm.at[sb[i2]], buf[i2], semg[i2])
            @pl.when(k > 0)
            def _():
                pltpu.make_async_copy(
                    dst_hbm.at[w, j, 0], db[i], semId[i]).wait()
            pltpu.sync_copy(buf[i], acc_sh.at[db[i]], add=True)
            if i < 2:
                pltpu.async_copy(dst_hbm.at[w, j + 3, 0], db[i], semId[i])
            else:
                @pl.when(k < NITER - 1)
                def _():
                    pltpu.async_copy(dst_hbm.at[w, j + 3, 0], db[i], semId[i])
        return 0
    lax.fori_loop(0, NITER, body, 0)

    jlast = NCHUNK - 2
    pltpu.make_async_copy(h_hbm.at[sb0], buf0, semg0).wait()
    pltpu.make_async_copy(dst_hbm.at[w, jlast, 0], db0, semId0).wait()
    pltpu.sync_copy(buf0, acc_sh.at[db0], add=True)
    pltpu.make_async_copy(h_hbm.at[sb1], buf1, semg1).wait()
    pltpu.make_async_copy(dst_hbm.at[w, jlast + 1, 0], db1, semId1).wait()
    pltpu.sync_copy(buf1, acc_sh.at[db1], add=True)
    plsc.subcore_barrier()

    pltpu.sync_copy(acc_sh.at[pl.ds(s * ROWS_PER_TILE, ROWS_PER_TILE)],
                    out_hbm.at[c, pl.ds(s * ROWS_PER_TILE, ROWS_PER_TILE)])


_R = 1000


def _tc0_body(hraw_ref, deg_ref, h_ref, ns_ref, nd_ref):
    degs = deg_ref[...]
    d_out = degs[0, :, 0:1] + degs[1, :, 0:1]
    d_in = degs[0, :, 8:9] + degs[1, :, 8:9]
    ns = jnp.where(d_out > 0, lax.rsqrt(jnp.maximum(d_out, 1.0)), 0.0)
    nd = jnp.where(d_in > 0, lax.rsqrt(jnp.maximum(d_in, 1.0)), 0.0)
    h_ref[...] = hraw_ref[...] * ns
    ns_ref[...] = ns
    nd_ref[...] = nd


def _tc_mm_body(x_ref, w_ref, o_ref):
    o_ref[...] = jnp.dot(x_ref[...], w_ref[...],
                        preferred_element_type=jnp.float32)


def _tc_mid_body(p_ref, nd_ref, b_ref, w_ref, ns_ref, o_ref):
    z = (p_ref[0] + p_ref[1]) * nd_ref[...] + b_ref[...]
    h = jnp.maximum(z, 0.0)
    o_ref[...] = jnp.dot(h, w_ref[...],
                         preferred_element_type=jnp.float32) * ns_ref[...]


def _tc_fin_body(p_ref, nd_ref, b_ref, o_ref):
    o_ref[...] = (p_ref[0] + p_ref[1]) * nd_ref[...] + b_ref[...]


_tc_mm = pl.pallas_call(
    _tc_mm_body,
    grid=(N // _R,),
    in_specs=[
        pl.BlockSpec((_R, D), lambda j: (j, 0)),
        pl.BlockSpec((D, D), lambda j: (0, 0)),
    ],
    out_specs=pl.BlockSpec((_R, D), lambda j: (j, 0)),
    out_shape=jax.ShapeDtypeStruct((N, D), jnp.float32),
)

_tc0 = pl.pallas_call(
    _tc0_body,
    grid=(N // _R,),
    in_specs=[
        pl.BlockSpec((_R, D), lambda j: (j, 0)),
        pl.BlockSpec((NC, _R, 16), lambda j: (0, j, 0)),
    ],
    out_specs=[
        pl.BlockSpec((_R, D), lambda j: (j, 0)),
        pl.BlockSpec((_R, 1), lambda j: (j, 0)),
        pl.BlockSpec((_R, 1), lambda j: (j, 0)),
    ],
    out_shape=[
        jax.ShapeDtypeStruct((N, D), jnp.float32),
        jax.ShapeDtypeStruct((N, 1), jnp.float32),
        jax.ShapeDtypeStruct((N, 1), jnp.float32),
    ],
)

_tc_mid = pl.pallas_call(
    _tc_mid_body,
    grid=(N // _R,),
    in_specs=[
        pl.BlockSpec((NC, _R, D), lambda j: (0, j, 0)),
        pl.BlockSpec((_R, 1), lambda j: (j, 0)),
        pl.BlockSpec((1, D), lambda j: (0, 0)),
        pl.BlockSpec((D, D), lambda j: (0, 0)),
        pl.BlockSpec((_R, 1), lambda j: (j, 0)),
    ],
    out_specs=pl.BlockSpec((_R, D), lambda j: (j, 0)),
    out_shape=jax.ShapeDtypeStruct((N, D), jnp.float32),
)

_tc_fin = pl.pallas_call(
    _tc_fin_body,
    grid=(N // _R,),
    in_specs=[
        pl.BlockSpec((NC, _R, D), lambda j: (0, j, 0)),
        pl.BlockSpec((_R, 1), lambda j: (j, 0)),
        pl.BlockSpec((1, D), lambda j: (0, 0)),
    ],
    out_specs=pl.BlockSpec((_R, D), lambda j: (j, 0)),
    out_shape=jax.ShapeDtypeStruct((N, D), jnp.float32),
)


def kernel(x, edge_index, W_in, b_in, W_hid, b_hid, W_out, b_out):
    src_i32 = edge_index[0].astype(jnp.int32)
    dst_i32 = edge_index[1].astype(jnp.int32)
    src3 = src_i32.reshape(NW, NCHUNK, 1, CH)
    dst3 = dst_i32.reshape(NW, NCHUNK, 1, CH)
    deg = _deg_kernel(src_i32.reshape(NW, NCHD, 1, CHD),
                      dst_i32.reshape(NW, NCHD, 1, CHD)).transpose(0, 2, 1)
    h1, ns, nd = _tc0(_tc_mm(x, W_in), deg)
    p1 = _msg_kernel(h1, src3, dst3)
    h2 = _tc_mid(p1, nd, b_in.reshape(1, D), W_hid, ns)
    p2 = _msg_kernel(h2, src3, dst3)
    h3 = _tc_mid(p2, nd, b_hid.reshape(1, D), W_out, ns)
    p3 = _msg_kernel(h3, src3, dst3)
    return _tc_fin(p3, nd, b_out.reshape(1, D))
